# Initial kernel scaffold; baseline (speedup 1.0000x reference)
import jax
import jax.numpy as jnp
from jax.experimental import pallas as pl


def kernel(x, dim):
    raise NotImplementedError("write your pallas kernel here")


# speedup vs baseline: 2.1262x; 2.1262x over previous
import functools

import jax
import jax.numpy as jnp
from jax import lax
from jax.experimental import pallas as pl
from jax.experimental.pallas import tpu as pltpu
from jax.experimental.pallas import tpu_sc as plsc

_PERIOD = 4
_START = 0
_NC = 2
_NS = 16
_NW = _NC * _NS

_CHUNK = 16


@functools.partial(jax.jit, static_argnames=("tot_rows", "d", "nchunks"))
def _sc_decimate(x_flat, idx, tot_rows, d, nchunks):
    mesh = plsc.VectorSubcoreMesh(
        core_axis_name="c", subcore_axis_name="s",
        num_cores=_NC, num_subcores=_NS,
    )
    rows_per_w = tot_rows // _NW

    @functools.partial(
        pl.kernel,
        out_type=jax.ShapeDtypeStruct((tot_rows, d), jnp.float32),
        mesh=mesh,
        scratch_types=[
            pltpu.VMEM((nchunks, _CHUNK), jnp.int32),
            pltpu.VMEM((_CHUNK, d), jnp.float32),
            pltpu.SemaphoreType.DMA,
        ],
    )
    def run(x_hbm, idx_hbm, out_hbm, idx_v, buf, sem):
        wid = lax.axis_index("s") * _NC + lax.axis_index("c")
        pltpu.sync_copy(idx_hbm.at[wid], idx_v)
        base = wid * rows_per_w

        def step(j, carry):
            pltpu.async_copy(x_hbm.at[idx_v.at[j]], buf, sem).wait()
            pltpu.sync_copy(buf, out_hbm.at[pl.ds(base + j * _CHUNK, _CHUNK)])
            return carry

        lax.fori_loop(0, nchunks, step, 0)

    return run(x_flat, idx)


def kernel(x, dim):
    b, n, d = x.shape
    off = jnp.asarray(dim, dtype=jnp.int32) - 1
    r_out = (n - _START + _PERIOD - 1) // _PERIOD
    tot_rows = b * r_out
    idx = (
        (jnp.arange(b, dtype=jnp.int32) * n)[:, None]
        + (_START + off + _PERIOD * jnp.arange(r_out, dtype=jnp.int32))[None, :]
    )
    nchunks = tot_rows // _NW // _CHUNK
    idx = idx.reshape(_NW, nchunks, _CHUNK)
    x_flat = x.reshape(b * n, d)
    out_flat = _sc_decimate(x_flat, idx, tot_rows, d, nchunks)
    return out_flat.reshape(b, r_out, d)


# speedup vs baseline: 2.4300x; 1.1429x over previous
import functools

import jax
import jax.numpy as jnp
from jax import lax
from jax.experimental import pallas as pl
from jax.experimental.pallas import tpu as pltpu
from jax.experimental.pallas import tpu_sc as plsc

_PERIOD = 4
_START = 0
_NC = 2
_NS = 16
_NW = _NC * _NS

_CHUNK = 16


@functools.partial(jax.jit, static_argnames=("tot_rows", "d", "nchunks"))
def _sc_decimate(x_flat, idx, tot_rows, d, nchunks):
    mesh = plsc.VectorSubcoreMesh(
        core_axis_name="c", subcore_axis_name="s",
        num_cores=_NC, num_subcores=_NS,
    )
    rows_per_w = tot_rows // _NW

    @functools.partial(
        pl.kernel,
        out_type=jax.ShapeDtypeStruct((tot_rows, d), jnp.float32),
        mesh=mesh,
        scratch_types=[
            pltpu.VMEM((nchunks, _CHUNK), jnp.int32),
            pltpu.VMEM((_CHUNK, d), jnp.float32),
            pltpu.VMEM((_CHUNK, d), jnp.float32),
            pltpu.SemaphoreType.DMA,
            pltpu.SemaphoreType.DMA,
        ],
    )
    def run(x_hbm, idx_hbm, out_hbm, idx_v, buf0, buf1, gsem, ssem):
        wid = lax.axis_index("s") * _NC + lax.axis_index("c")
        pltpu.sync_copy(idx_hbm.at[wid], idx_v)
        base = wid * rows_per_w
        bufs = (buf0, buf1)

        gathers = [None] * nchunks
        scatters = [None] * nchunks
        gathers[0] = pltpu.async_copy(x_hbm.at[idx_v.at[0]], bufs[0], gsem)
        for j in range(nchunks):
            cur = bufs[j % 2]
            if j + 1 < nchunks:
                nxt = bufs[(j + 1) % 2]
                if j >= 1:
                    scatters[j - 1].wait()
                gathers[j + 1] = pltpu.async_copy(
                    x_hbm.at[idx_v.at[j + 1]], nxt, gsem)
            gathers[j].wait()
            scatters[j] = pltpu.async_copy(
                cur, out_hbm.at[pl.ds(base + j * _CHUNK, _CHUNK)], ssem)
        if nchunks >= 2:
            scatters[nchunks - 2].wait()
        scatters[nchunks - 1].wait()

    return run(x_flat, idx)


def kernel(x, dim):
    b, n, d = x.shape
    off = jnp.asarray(dim, dtype=jnp.int32) - 1
    r_out = (n - _START + _PERIOD - 1) // _PERIOD
    tot_rows = b * r_out
    idx = (
        (jnp.arange(b, dtype=jnp.int32) * n)[:, None]
        + (_START + off + _PERIOD * jnp.arange(r_out, dtype=jnp.int32))[None, :]
    )
    nchunks = tot_rows // _NW // _CHUNK
    idx = idx.reshape(_NW, nchunks, _CHUNK)
    x_flat = x.reshape(b * n, d)
    out_flat = _sc_decimate(x_flat, idx, tot_rows, d, nchunks)
    return out_flat.reshape(b, r_out, d)
